# baseline (speedup 1.0000x reference)
import functools

import jax
import jax.numpy as jnp
from jax import lax
from jax.experimental import pallas as pl
from jax.experimental.pallas import tpu as pltpu
from jax.experimental.pallas import tpu_sc as plsc

_TAU = 1.0
_B, _N, _M = 64, 512, 512
_NM = _N * _M

_TILES = 32
_BATCH_PER_TILE = _B // _TILES
_ZROWS = 64
_NCOPIES = _N // _ZROWS


@functools.partial(
    pl.kernel,
    out_type=jax.ShapeDtypeStruct((_B, _N, _M), jnp.float32),
    mesh=plsc.VectorSubcoreMesh(core_axis_name="c", subcore_axis_name="s"),
    scratch_types=[
        pltpu.VMEM((_ZROWS, _M), jnp.float32),
        pltpu.SemaphoreType.DMA,
    ],
)
def _sc_fill(out_hbm, zbuf, sem):
    wid = lax.axis_index("s") * 2 + lax.axis_index("c")

    def _zero_body(i, carry):
        zbuf[i // (_M // 16), pl.ds((i % (_M // 16)) * 16, 16)] = jnp.zeros(
            (16,), jnp.float32
        )
        return carry

    lax.fori_loop(0, _ZROWS * _M // 16, _zero_body, 0)

    for j in range(_BATCH_PER_TILE):
        b = wid * _BATCH_PER_TILE + j
        for k in range(_NCOPIES):
            pltpu.async_copy(
                zbuf, out_hbm.at[b, pl.ds(k * _ZROWS, _ZROWS), :], sem
            ).wait()


_BB = 8


def _argmax_body(x_ref, o_ref, ir_ref, ic_ref, idx_ref):
    b = pl.program_id(0)
    e = jnp.exp(-x_ref[...] * (1.0 / _TAU)) * o_ref[...]
    big = jnp.int32(2**31 - 1)
    for j in range(_BB):
        ej = e[j : j + 1]
        m = jnp.max(ej)
        rid = jnp.min(jnp.where(ej == m, ic_ref[...], big), axis=1)
        flat = jnp.min(jnp.where(rid < big, rid * _M + ir_ref[...], big))
        idx_ref[j, 0, 0] = (b * _BB + j) * _NM + flat


def _scatter_body(idxs_ref, idxv_ref, filled_ref, out_ref, rows_ref, sem):
    del filled_ref
    cols = lax.rem(idxv_ref[...], jnp.int32(_M))
    ciota = jax.lax.broadcasted_iota(jnp.int32, (_B, _M), 1)
    rows_ref[...] = (ciota == cols).astype(jnp.float32)
    copies = []
    for b in range(_B):
        f = idxs_ref[b, 0, 0]
        r = (f // _M) - b * _N
        copies.append(
            pltpu.make_async_copy(
                rows_ref.at[pl.ds(b, 1), :],
                out_ref.at[b, pl.ds(r, 1), :],
                sem,
            )
        )
    for cp in copies:
        cp.start()
    for cp in copies:
        cp.wait()


def kernel(x, o):
    filled = _sc_fill()

    ir = jnp.arange(_M, dtype=jnp.int32).reshape(1, _M)
    ic = jnp.arange(_N, dtype=jnp.int32).reshape(1, _N, 1)

    idx = pl.pallas_call(
        _argmax_body,
        grid=(_B // _BB,),
        in_specs=[
            pl.BlockSpec((_BB, _N, _M), lambda b: (b, 0, 0)),
            pl.BlockSpec((_BB, _N, _M), lambda b: (b, 0, 0)),
            pl.BlockSpec((1, _M), lambda b: (0, 0)),
            pl.BlockSpec((1, _N, 1), lambda b: (0, 0, 0)),
        ],
        out_specs=pl.BlockSpec(
            (_BB, 1, 1), lambda b: (b, 0, 0), memory_space=pltpu.SMEM
        ),
        out_shape=jax.ShapeDtypeStruct((_B, 1, 1), jnp.int32),
    )(x, o, ir, ic)

    out = pl.pallas_call(
        _scatter_body,
        in_specs=[
            pl.BlockSpec(memory_space=pltpu.SMEM),
            pl.BlockSpec(memory_space=pltpu.VMEM),
            pl.BlockSpec(memory_space=pl.ANY),
        ],
        out_specs=pl.BlockSpec(memory_space=pl.ANY),
        out_shape=jax.ShapeDtypeStruct((_B, _N, _M), jnp.float32),
        scratch_shapes=[
            pltpu.VMEM((_B, _M), jnp.float32),
            pltpu.SemaphoreType.DMA,
        ],
        input_output_aliases={2: 0},
    )(idx, idx.reshape(_B, 1), filled)

    return out


# speedup vs baseline: 1.0096x; 1.0096x over previous
import functools

import numpy as np

import jax
import jax.numpy as jnp
from jax import lax
from jax.experimental import pallas as pl
from jax.experimental.pallas import tpu as pltpu
from jax.experimental.pallas import tpu_sc as plsc

_TAU = 1.0
_B, _N, _M = 64, 512, 512
_NM = _N * _M

_TILES = 32
_BATCH_PER_TILE = _B // _TILES
_ZROWS = 64
_NCOPIES = _N // _ZROWS


@functools.partial(
    pl.kernel,
    out_type=jax.ShapeDtypeStruct((_B, _N, _M), jnp.float32),
    mesh=plsc.VectorSubcoreMesh(core_axis_name="c", subcore_axis_name="s"),
    scratch_types=[
        pltpu.VMEM((_ZROWS, _M), jnp.float32),
        pltpu.SemaphoreType.DMA,
    ],
)
def _sc_fill(out_hbm, zbuf, sem):
    wid = lax.axis_index("s") * 2 + lax.axis_index("c")

    def _zero_body(i, carry):
        zbuf[i // (_M // 16), pl.ds((i % (_M // 16)) * 16, 16)] = jnp.zeros(
            (16,), jnp.float32
        )
        return carry

    lax.fori_loop(0, _ZROWS * _M // 16, _zero_body, 0)

    for j in range(_BATCH_PER_TILE):
        b = wid * _BATCH_PER_TILE + j
        for k in range(_NCOPIES):
            pltpu.async_copy(
                zbuf, out_hbm.at[b, pl.ds(k * _ZROWS, _ZROWS), :], sem
            ).wait()


_BB = 8


def _argmax_body(x_ref, o_ref, ir_ref, ic_ref, idx_ref):
    b = pl.program_id(0)
    e = jnp.exp(-x_ref[...] * (1.0 / _TAU)) * o_ref[...]
    big = jnp.int32(2**31 - 1)
    for j in range(_BB):
        ej = e[j : j + 1]
        m = jnp.max(ej)
        rid = jnp.min(jnp.where(ej == m, ic_ref[...], big), axis=1)
        flat = jnp.min(jnp.where(rid < big, rid * _M + ir_ref[...], big))
        idx_ref[j, 0, 0] = (b * _BB + j) * _NM + flat


def _scatter_body(idxs_ref, idxv_ref, filled_ref, out_ref, rows_ref, sem):
    del filled_ref
    cols = lax.rem(idxv_ref[...], jnp.int32(_M))
    ciota = jax.lax.broadcasted_iota(jnp.int32, (_B, _M), 1)
    rows_ref[...] = (ciota == cols).astype(jnp.float32)
    copies = []
    for b in range(_B):
        f = idxs_ref[b, 0, 0]
        r = (f // _M) - b * _N
        copies.append(
            pltpu.make_async_copy(
                rows_ref.at[pl.ds(b, 1), :],
                out_ref.at[b, pl.ds(r, 1), :],
                sem,
            )
        )
    for cp in copies:
        cp.start()
    for cp in copies:
        cp.wait()


def kernel(x, o):
    filled = _sc_fill()

    ir = np.arange(_M, dtype=np.int32).reshape(1, _M)
    ic = np.arange(_N, dtype=np.int32).reshape(1, _N, 1)

    idx = pl.pallas_call(
        _argmax_body,
        grid=(_B // _BB,),
        in_specs=[
            pl.BlockSpec((_BB, _N, _M), lambda b: (b, 0, 0)),
            pl.BlockSpec((_BB, _N, _M), lambda b: (b, 0, 0)),
            pl.BlockSpec((1, _M), lambda b: (0, 0)),
            pl.BlockSpec((1, _N, 1), lambda b: (0, 0, 0)),
        ],
        out_specs=pl.BlockSpec(
            (_BB, 1, 1), lambda b: (b, 0, 0), memory_space=pltpu.SMEM
        ),
        out_shape=jax.ShapeDtypeStruct((_B, 1, 1), jnp.int32),
    )(x, o, ir, ic)

    out = pl.pallas_call(
        _scatter_body,
        in_specs=[
            pl.BlockSpec(memory_space=pltpu.SMEM),
            pl.BlockSpec(memory_space=pltpu.VMEM),
            pl.BlockSpec(memory_space=pl.ANY),
        ],
        out_specs=pl.BlockSpec(memory_space=pl.ANY),
        out_shape=jax.ShapeDtypeStruct((_B, _N, _M), jnp.float32),
        scratch_shapes=[
            pltpu.VMEM((_B, _M), jnp.float32),
            pltpu.SemaphoreType.DMA,
        ],
        input_output_aliases={2: 0},
    )(idx, idx.reshape(_B, 1), filled)

    return out


# speedup vs baseline: 1.2313x; 1.2196x over previous
import functools

import numpy as np

import jax
import jax.numpy as jnp
from jax import lax
from jax.experimental import pallas as pl
from jax.experimental.pallas import tpu as pltpu
from jax.experimental.pallas import tpu_sc as plsc

_TAU = 1.0
_B, _N, _M = 64, 512, 512
_NM = _N * _M

_TILES = 32
_BATCH_PER_TILE = _B // _TILES
_ZROWS = 64
_NCOPIES = _N // _ZROWS


@functools.partial(
    pl.kernel,
    out_type=jax.ShapeDtypeStruct((_B, _N, _M), jnp.float32),
    mesh=plsc.VectorSubcoreMesh(core_axis_name="c", subcore_axis_name="s"),
    scratch_types=[
        pltpu.VMEM((_ZROWS, _M), jnp.float32),
        pltpu.SemaphoreType.DMA,
    ],
)
def _sc_fill(out_hbm, zbuf, sem):
    wid = lax.axis_index("s") * 2 + lax.axis_index("c")

    def _zero_body(i, carry):
        zbuf[i // (_M // 16), pl.ds((i % (_M // 16)) * 16, 16)] = jnp.zeros(
            (16,), jnp.float32
        )
        return carry

    lax.fori_loop(0, _ZROWS * _M // 16, _zero_body, 0)

    for j in range(_BATCH_PER_TILE):
        b = wid * _BATCH_PER_TILE + j
        for k in range(_NCOPIES):
            pltpu.async_copy(
                zbuf, out_hbm.at[b, pl.ds(k * _ZROWS, _ZROWS), :], sem
            ).wait()


_BB = 8


def _argmax_body(x_ref, o_ref, ir_ref, ic_ref, idx_ref):
    b = pl.program_id(0)
    e = jnp.exp(-x_ref[...] * (1.0 / _TAU)) * o_ref[...]
    big = jnp.int32(2**31 - 1)
    for j in range(_BB):
        ej = e[j : j + 1]
        m = jnp.max(ej)
        rid = jnp.min(jnp.where(ej == m, ic_ref[...], big), axis=1)
        flat = jnp.min(jnp.where(rid < big, rid * _M + ir_ref[...], big))
        idx_ref[j, 0, 0] = (b * _BB + j) * _NM + flat


def _scatter_body(idxs_ref, idxv_ref, filled_ref, out_ref, rows_ref, sem):
    del filled_ref
    cols = lax.rem(idxv_ref[...], jnp.int32(_M))
    ciota = jax.lax.broadcasted_iota(jnp.int32, (_B, _M), 1)
    rows_ref[...] = (ciota == cols).astype(jnp.float32)
    copies = []
    for b in range(_B):
        f = idxs_ref[b, 0, 0]
        r = (f // _M) - b * _N
        copies.append(
            pltpu.make_async_copy(
                rows_ref.at[pl.ds(b, 1), :],
                out_ref.at[b, pl.ds(r, 1), :],
                sem,
            )
        )
    for cp in copies:
        cp.start()
    for cp in copies:
        cp.wait()


_FBB = 4


def _fused_body(x_ref, o_ref, ir_ref, ic_ref, out_ref):
    e = jnp.exp(-x_ref[...] * (1.0 / _TAU)) * o_ref[...]
    big = jnp.int32(2**31 - 1)
    for j in range(_FBB):
        ej = e[j : j + 1]
        m = jnp.max(ej)
        rid = jnp.min(jnp.where(ej == m, ic_ref[...], big), axis=1)
        flat = jnp.min(jnp.where(rid < big, rid * _M + ir_ref[0], big))
        r = flat // _M
        c = flat - r * _M
        onehot = (ic_ref[...] == r) & (ir_ref[...] == c)
        out_ref[pl.ds(j, 1)] = onehot.astype(jnp.float32)


def kernel(x, o):
    ir = np.arange(_M, dtype=np.int32).reshape(1, 1, _M)
    ic = np.arange(_N, dtype=np.int32).reshape(1, _N, 1)
    return pl.pallas_call(
        _fused_body,
        grid=(_B // _FBB,),
        in_specs=[
            pl.BlockSpec((_FBB, _N, _M), lambda b: (b, 0, 0)),
            pl.BlockSpec((_FBB, _N, _M), lambda b: (b, 0, 0)),
            pl.BlockSpec((1, 1, _M), lambda b: (0, 0, 0)),
            pl.BlockSpec((1, _N, 1), lambda b: (0, 0, 0)),
        ],
        out_specs=pl.BlockSpec((_FBB, _N, _M), lambda b: (b, 0, 0)),
        out_shape=jax.ShapeDtypeStruct((_B, _N, _M), jnp.float32),
    )(x, o, ir, ic)


def _unused_kernel(x, o):
    filled = _sc_fill()

    ir = np.arange(_M, dtype=np.int32).reshape(1, _M)
    ic = np.arange(_N, dtype=np.int32).reshape(1, _N, 1)

    idx = pl.pallas_call(
        _argmax_body,
        grid=(_B // _BB,),
        in_specs=[
            pl.BlockSpec((_BB, _N, _M), lambda b: (b, 0, 0)),
            pl.BlockSpec((_BB, _N, _M), lambda b: (b, 0, 0)),
            pl.BlockSpec((1, _M), lambda b: (0, 0)),
            pl.BlockSpec((1, _N, 1), lambda b: (0, 0, 0)),
        ],
        out_specs=pl.BlockSpec(
            (_BB, 1, 1), lambda b: (b, 0, 0), memory_space=pltpu.SMEM
        ),
        out_shape=jax.ShapeDtypeStruct((_B, 1, 1), jnp.int32),
    )(x, o, ir, ic)

    out = pl.pallas_call(
        _scatter_body,
        in_specs=[
            pl.BlockSpec(memory_space=pltpu.SMEM),
            pl.BlockSpec(memory_space=pltpu.VMEM),
            pl.BlockSpec(memory_space=pl.ANY),
        ],
        out_specs=pl.BlockSpec(memory_space=pl.ANY),
        out_shape=jax.ShapeDtypeStruct((_B, _N, _M), jnp.float32),
        scratch_shapes=[
            pltpu.VMEM((_B, _M), jnp.float32),
            pltpu.SemaphoreType.DMA,
        ],
        input_output_aliases={2: 0},
    )(idx, idx.reshape(_B, 1), filled)

    return out


# speedup vs baseline: 1.2891x; 1.0470x over previous
import functools

import numpy as np

import jax
import jax.numpy as jnp
from jax import lax
from jax.experimental import pallas as pl
from jax.experimental.pallas import tpu as pltpu
from jax.experimental.pallas import tpu_sc as plsc

_TAU = 1.0
_B, _N, _M = 64, 512, 512
_NM = _N * _M

_TILES = 32
_BATCH_PER_TILE = _B // _TILES
_ZROWS = 64
_NCOPIES = _N // _ZROWS


@functools.partial(
    pl.kernel,
    out_type=jax.ShapeDtypeStruct((_B, _N, _M), jnp.float32),
    mesh=plsc.VectorSubcoreMesh(core_axis_name="c", subcore_axis_name="s"),
    scratch_types=[
        pltpu.VMEM((_ZROWS, _M), jnp.float32),
        pltpu.SemaphoreType.DMA,
    ],
)
def _sc_fill(out_hbm, zbuf, sem):
    wid = lax.axis_index("s") * 2 + lax.axis_index("c")

    def _zero_body(i, carry):
        zbuf[i // (_M // 16), pl.ds((i % (_M // 16)) * 16, 16)] = jnp.zeros(
            (16,), jnp.float32
        )
        return carry

    lax.fori_loop(0, _ZROWS * _M // 16, _zero_body, 0)

    for j in range(_BATCH_PER_TILE):
        b = wid * _BATCH_PER_TILE + j
        for k in range(_NCOPIES):
            pltpu.async_copy(
                zbuf, out_hbm.at[b, pl.ds(k * _ZROWS, _ZROWS), :], sem
            ).wait()


_BB = 8


def _argmax_body(x_ref, o_ref, ir_ref, ic_ref, idx_ref):
    b = pl.program_id(0)
    e = jnp.exp(-x_ref[...] * (1.0 / _TAU)) * o_ref[...]
    big = jnp.int32(2**31 - 1)
    for j in range(_BB):
        ej = e[j : j + 1]
        m = jnp.max(ej)
        rid = jnp.min(jnp.where(ej == m, ic_ref[...], big), axis=1)
        flat = jnp.min(jnp.where(rid < big, rid * _M + ir_ref[...], big))
        idx_ref[j, 0, 0] = (b * _BB + j) * _NM + flat


def _scatter_body(idxs_ref, idxv_ref, filled_ref, out_ref, rows_ref, sem):
    del filled_ref
    cols = lax.rem(idxv_ref[...], jnp.int32(_M))
    ciota = jax.lax.broadcasted_iota(jnp.int32, (_B, _M), 1)
    rows_ref[...] = (ciota == cols).astype(jnp.float32)
    copies = []
    for b in range(_B):
        f = idxs_ref[b, 0, 0]
        r = (f // _M) - b * _N
        copies.append(
            pltpu.make_async_copy(
                rows_ref.at[pl.ds(b, 1), :],
                out_ref.at[b, pl.ds(r, 1), :],
                sem,
            )
        )
    for cp in copies:
        cp.start()
    for cp in copies:
        cp.wait()


_FBB = 8


def _fused_body(x_ref, o_ref, ir_ref, ic_ref, out_ref):
    e = jnp.exp(-x_ref[...] * (1.0 / _TAU)) * o_ref[...]
    big = jnp.int32(2**31 - 1)
    for j in range(_FBB):
        ej = e[j : j + 1]
        m = jnp.max(ej)
        rid = jnp.min(jnp.where(ej == m, ic_ref[...], big), axis=1)
        flat = jnp.min(jnp.where(rid < big, rid * _M + ir_ref[0], big))
        r = flat // _M
        c = flat - r * _M
        onehot = (ic_ref[...] == r) & (ir_ref[...] == c)
        out_ref[pl.ds(j, 1)] = onehot.astype(jnp.float32)


def kernel(x, o):
    ir = np.arange(_M, dtype=np.int32).reshape(1, 1, _M)
    ic = np.arange(_N, dtype=np.int32).reshape(1, _N, 1)
    return pl.pallas_call(
        _fused_body,
        grid=(_B // _FBB,),
        in_specs=[
            pl.BlockSpec((_FBB, _N, _M), lambda b: (b, 0, 0)),
            pl.BlockSpec((_FBB, _N, _M), lambda b: (b, 0, 0)),
            pl.BlockSpec((1, 1, _M), lambda b: (0, 0, 0)),
            pl.BlockSpec((1, _N, 1), lambda b: (0, 0, 0)),
        ],
        out_specs=pl.BlockSpec((_FBB, _N, _M), lambda b: (b, 0, 0)),
        out_shape=jax.ShapeDtypeStruct((_B, _N, _M), jnp.float32),
    )(x, o, ir, ic)


def _unused_kernel(x, o):
    filled = _sc_fill()

    ir = np.arange(_M, dtype=np.int32).reshape(1, _M)
    ic = np.arange(_N, dtype=np.int32).reshape(1, _N, 1)

    idx = pl.pallas_call(
        _argmax_body,
        grid=(_B // _BB,),
        in_specs=[
            pl.BlockSpec((_BB, _N, _M), lambda b: (b, 0, 0)),
            pl.BlockSpec((_BB, _N, _M), lambda b: (b, 0, 0)),
            pl.BlockSpec((1, _M), lambda b: (0, 0)),
            pl.BlockSpec((1, _N, 1), lambda b: (0, 0, 0)),
        ],
        out_specs=pl.BlockSpec(
            (_BB, 1, 1), lambda b: (b, 0, 0), memory_space=pltpu.SMEM
        ),
        out_shape=jax.ShapeDtypeStruct((_B, 1, 1), jnp.int32),
    )(x, o, ir, ic)

    out = pl.pallas_call(
        _scatter_body,
        in_specs=[
            pl.BlockSpec(memory_space=pltpu.SMEM),
            pl.BlockSpec(memory_space=pltpu.VMEM),
            pl.BlockSpec(memory_space=pl.ANY),
        ],
        out_specs=pl.BlockSpec(memory_space=pl.ANY),
        out_shape=jax.ShapeDtypeStruct((_B, _N, _M), jnp.float32),
        scratch_shapes=[
            pltpu.VMEM((_B, _M), jnp.float32),
            pltpu.SemaphoreType.DMA,
        ],
        input_output_aliases={2: 0},
    )(idx, idx.reshape(_B, 1), filled)

    return out
